# baseline (speedup 1.0000x reference)
import functools

import jax
import jax.numpy as jnp
import numpy as np
from jax.experimental import pallas as pl

N_V = 54
N_E = 72
K = 3
DV = 32
DE = 16
L = 5
P = 4
BB = 256


def _dot(x, w):
    r, c, kf = x.shape
    y = jax.lax.dot_general(
        x.astype(w.dtype).reshape(r * c, kf), w,
        dimension_numbers=(((1,), (0,)), ((), ())),
        preferred_element_type=jnp.float32,
    )
    return y.reshape(r, c, w.shape[1])


def _sage_block(vt_ref, et_ref, wp_ref, we_ref, wh1_ref, wh2_ref,
                ebp_ref, hbp_ref, ones_ref, out_ref):
    vf = vt_ref[...]
    et = et_ref[...]
    ep = jnp.concatenate(
        [jnp.concatenate([et[3 * w + j:3 * w + j + 1] for w in range(24)],
                         axis=0) for j in range(K)], axis=0)
    ones_bd = ones_ref[...]

    for i in range(L):
        p = _dot(vf, wp_ref[i])
        t = _dot(ep, we_ref[i])
        p_ext = jnp.concatenate([p, p[:3]], axis=0)
        pre = None
        for j in range(K):
            s = j + 1
            rolled = p_ext[s:s + N_V]
            tj = t[24 * j:24 * (j + 1)]
            tj54 = jnp.concatenate([tj, tj, tj[:6]], axis=0)
            x = rolled + tj54
            pre = x if pre is None else jnp.maximum(pre, x)
        agg = jnp.tanh(pre + ebp_ref[i])
        vf = _dot(vf, wh1_ref[i]) + _dot(agg, wh2_ref[i]) + hbp_ref[i]
        if i < L - 1:
            vf = jnp.tanh(vf)
            ss = _dot(vf * vf, ones_bd)
            vf = vf * jax.lax.rsqrt(ss)
    out_ref[...] = vf


def _kron4(w):
    l, f, o = w.shape
    eye = jnp.eye(P, dtype=w.dtype)
    return (eye[None, :, None, :, None] * w[:, None, :, None, :]).reshape(
        l, P * f, P * o)


NCHUNK = 1


def kernel(vertices, edges, eW, eb, hW, hb):
    B = vertices.shape[0]

    eWv = eW[:, :DV, :]
    eWe = eW[:, DV:, :]
    wp = _kron4(eWv).astype(jnp.bfloat16)
    we = _kron4(eWe).astype(jnp.bfloat16)
    wh1 = _kron4(hW[:, :DV, :]).astype(jnp.bfloat16)
    wh2 = _kron4(hW[:, DV:, :]).astype(jnp.bfloat16)
    ebp = jnp.tile(eb, (1, P)).reshape(L, 1, P * 32)
    hbp = jnp.tile(hb, (1, P)).reshape(L, 1, P * 32)
    ones_bd = jnp.kron(jnp.eye(P, dtype=jnp.bfloat16),
                       jnp.ones((DV, DV), jnp.bfloat16))

    def run_chunk(v_c, e_c):
        bc = v_c.shape[0]
        b4 = bc // P
        vt = v_c.astype(jnp.bfloat16).transpose(1, 0, 2).reshape(
            N_V, b4, P * DV)
        et = e_c.astype(jnp.bfloat16).transpose(1, 0, 2).reshape(
            N_E, b4, P * DE)
        out = pl.pallas_call(
            _sage_block,
            grid=(b4 // BB,),
            in_specs=[
                pl.BlockSpec((N_V, BB, P * DV), lambda b: (0, b, 0)),
                pl.BlockSpec((K * 24, BB, P * DE), lambda b: (0, b, 0)),
                pl.BlockSpec((L, P * DV, P * DV), lambda b: (0, 0, 0)),
                pl.BlockSpec((L, P * DE, P * DV), lambda b: (0, 0, 0)),
                pl.BlockSpec((L, P * DV, P * DV), lambda b: (0, 0, 0)),
                pl.BlockSpec((L, P * DV, P * DV), lambda b: (0, 0, 0)),
                pl.BlockSpec((L, 1, P * 32), lambda b: (0, 0, 0)),
                pl.BlockSpec((L, 1, P * 32), lambda b: (0, 0, 0)),
                pl.BlockSpec((P * DV, P * DV), lambda b: (0, 0)),
            ],
            out_specs=pl.BlockSpec((N_V, BB, P * DV), lambda b: (0, b, 0)),
            out_shape=jax.ShapeDtypeStruct((N_V, b4, P * DV), jnp.float32),
        )(vt, et, wp, we, wh1, wh2, ebp, hbp, ones_bd)
        return out.reshape(N_V, bc, DV).transpose(1, 0, 2)

    bc = B // NCHUNK
    outs = [run_chunk(vertices[k * bc:(k + 1) * bc],
                      edges[k * bc:(k + 1) * bc]) for k in range(NCHUNK)]
    return jnp.concatenate(outs, axis=0)


# speedup vs baseline: 1.0689x; 1.0689x over previous
import functools

import jax
import jax.numpy as jnp
import numpy as np
from jax.experimental import pallas as pl

N_V = 54
N_E = 72
K = 3
DV = 32
DE = 16
L = 5
P = 4
BB = 128


def _dot(x, w):
    r, c, kf = x.shape
    y = jax.lax.dot_general(
        x.astype(w.dtype).reshape(r * c, kf), w,
        dimension_numbers=(((1,), (0,)), ((), ())),
        preferred_element_type=jnp.float32,
    )
    return y.reshape(r, c, w.shape[1])


def _sage_block(vt_ref, et_ref, wp_ref, we_ref, wh1_ref, wh2_ref,
                ebp_ref, hbp_ref, ones_ref, out_ref):
    vf = vt_ref[...]
    et = et_ref[...]
    ep = jnp.concatenate(
        [jnp.concatenate([et[3 * w + j:3 * w + j + 1] for w in range(24)],
                         axis=0) for j in range(K)], axis=0)
    ones_bd = ones_ref[...]

    for i in range(L):
        p = _dot(vf, wp_ref[i])
        t = _dot(ep, we_ref[i])
        p_ext = jnp.concatenate([p, p[:3]], axis=0)
        pre = None
        for j in range(K):
            s = j + 1
            rolled = p_ext[s:s + N_V]
            tj = t[24 * j:24 * (j + 1)]
            tj54 = jnp.concatenate([tj, tj, tj[:6]], axis=0)
            x = rolled + tj54
            pre = x if pre is None else jnp.maximum(pre, x)
        agg = jnp.tanh(pre + ebp_ref[i])
        vf = _dot(vf, wh1_ref[i]) + _dot(agg, wh2_ref[i]) + hbp_ref[i]
        if i < L - 1:
            vf = jnp.tanh(vf)
            ss = _dot(vf * vf, ones_bd)
            vf = vf * jax.lax.rsqrt(ss)
    out_ref[...] = vf


def _kron4(w):
    l, f, o = w.shape
    eye = jnp.eye(P, dtype=w.dtype)
    return (eye[None, :, None, :, None] * w[:, None, :, None, :]).reshape(
        l, P * f, P * o)


NCHUNK = 1


def kernel(vertices, edges, eW, eb, hW, hb):
    B = vertices.shape[0]

    eWv = eW[:, :DV, :]
    eWe = eW[:, DV:, :]
    wp = _kron4(eWv).astype(jnp.bfloat16)
    we = _kron4(eWe).astype(jnp.bfloat16)
    wh1 = _kron4(hW[:, :DV, :]).astype(jnp.bfloat16)
    wh2 = _kron4(hW[:, DV:, :]).astype(jnp.bfloat16)
    ebp = jnp.tile(eb, (1, P)).reshape(L, 1, P * 32)
    hbp = jnp.tile(hb, (1, P)).reshape(L, 1, P * 32)
    ones_bd = jnp.kron(jnp.eye(P, dtype=jnp.bfloat16),
                       jnp.ones((DV, DV), jnp.bfloat16))

    def run_chunk(v_c, e_c):
        bc = v_c.shape[0]
        b4 = bc // P
        vt = v_c.astype(jnp.bfloat16).transpose(1, 0, 2).reshape(
            N_V, b4, P * DV)
        et = e_c.astype(jnp.bfloat16).transpose(1, 0, 2).reshape(
            N_E, b4, P * DE)
        out = pl.pallas_call(
            _sage_block,
            grid=(b4 // BB,),
            in_specs=[
                pl.BlockSpec((N_V, BB, P * DV), lambda b: (0, b, 0)),
                pl.BlockSpec((K * 24, BB, P * DE), lambda b: (0, b, 0)),
                pl.BlockSpec((L, P * DV, P * DV), lambda b: (0, 0, 0)),
                pl.BlockSpec((L, P * DE, P * DV), lambda b: (0, 0, 0)),
                pl.BlockSpec((L, P * DV, P * DV), lambda b: (0, 0, 0)),
                pl.BlockSpec((L, P * DV, P * DV), lambda b: (0, 0, 0)),
                pl.BlockSpec((L, 1, P * 32), lambda b: (0, 0, 0)),
                pl.BlockSpec((L, 1, P * 32), lambda b: (0, 0, 0)),
                pl.BlockSpec((P * DV, P * DV), lambda b: (0, 0)),
            ],
            out_specs=pl.BlockSpec((N_V, BB, P * DV), lambda b: (0, b, 0)),
            out_shape=jax.ShapeDtypeStruct((N_V, b4, P * DV), jnp.float32),
        )(vt, et, wp, we, wh1, wh2, ebp, hbp, ones_bd)
        return out.reshape(N_V, bc, DV).transpose(1, 0, 2)

    bc = B // NCHUNK
    outs = [run_chunk(vertices[k * bc:(k + 1) * bc],
                      edges[k * bc:(k + 1) * bc]) for k in range(NCHUNK)]
    return jnp.concatenate(outs, axis=0)


# speedup vs baseline: 1.0897x; 1.0195x over previous
import functools

import jax
import jax.numpy as jnp
import numpy as np
from jax.experimental import pallas as pl

N_V = 54
N_E = 72
K = 3
DV = 32
DE = 16
L = 5
P = 4
BB = 128


def _dot(x, w, out_dtype=jnp.float32):
    r, c, kf = x.shape
    y = jax.lax.dot_general(
        x.astype(w.dtype).reshape(r * c, kf), w,
        dimension_numbers=(((1,), (0,)), ((), ())),
        preferred_element_type=out_dtype,
    )
    return y.reshape(r, c, w.shape[1])


def _sage_block(vt_ref, et_ref, wp_ref, we_ref, wh1_ref, wh2_ref,
                ebp_ref, hbp_ref, ones_ref, out_ref):
    vf = vt_ref[...]
    et = et_ref[...]
    ep = jnp.concatenate(
        [jnp.concatenate([et[3 * w + j:3 * w + j + 1] for w in range(24)],
                         axis=0) for j in range(K)], axis=0)
    ones_bd = ones_ref[...]

    for i in range(L):
        p = _dot(vf, wp_ref[i]).astype(jnp.bfloat16)
        t = _dot(ep, we_ref[i]).astype(jnp.bfloat16)
        p_ext = jnp.concatenate([p, p[:3]], axis=0)
        pre = None
        for j in range(K):
            s = j + 1
            rolled = p_ext[s:s + N_V]
            tj = t[24 * j:24 * (j + 1)]
            tj54 = jnp.concatenate([tj, tj, tj[:6]], axis=0)
            x = rolled + tj54
            pre = x if pre is None else jnp.maximum(pre, x)
        agg = jnp.tanh(pre + ebp_ref[i])
        vf = _dot(vf, wh1_ref[i]) + _dot(agg, wh2_ref[i]) + hbp_ref[i]
        if i < L - 1:
            vf = jnp.tanh(vf)
            ss = _dot(vf * vf, ones_bd)
            vf = vf * jax.lax.rsqrt(ss)
    out_ref[...] = vf


def _kron4(w):
    l, f, o = w.shape
    eye = jnp.eye(P, dtype=w.dtype)
    return (eye[None, :, None, :, None] * w[:, None, :, None, :]).reshape(
        l, P * f, P * o)


NCHUNK = 1


def kernel(vertices, edges, eW, eb, hW, hb):
    B = vertices.shape[0]

    eWv = eW[:, :DV, :]
    eWe = eW[:, DV:, :]
    wp = _kron4(eWv).astype(jnp.bfloat16)
    we = _kron4(eWe).astype(jnp.bfloat16)
    wh1 = _kron4(hW[:, :DV, :]).astype(jnp.bfloat16)
    wh2 = _kron4(hW[:, DV:, :]).astype(jnp.bfloat16)
    ebp = jnp.tile(eb, (1, P)).reshape(L, 1, P * 32).astype(jnp.bfloat16)
    hbp = jnp.tile(hb, (1, P)).reshape(L, 1, P * 32)
    ones_bd = jnp.kron(jnp.eye(P, dtype=jnp.bfloat16),
                       jnp.ones((DV, DV), jnp.bfloat16))

    def run_chunk(v_c, e_c):
        bc = v_c.shape[0]
        b4 = bc // P
        vt = v_c.astype(jnp.bfloat16).transpose(1, 0, 2).reshape(
            N_V, b4, P * DV)
        et = e_c.astype(jnp.bfloat16).transpose(1, 0, 2).reshape(
            N_E, b4, P * DE)
        out = pl.pallas_call(
            _sage_block,
            grid=(b4 // BB,),
            in_specs=[
                pl.BlockSpec((N_V, BB, P * DV), lambda b: (0, b, 0)),
                pl.BlockSpec((K * 24, BB, P * DE), lambda b: (0, b, 0)),
                pl.BlockSpec((L, P * DV, P * DV), lambda b: (0, 0, 0)),
                pl.BlockSpec((L, P * DE, P * DV), lambda b: (0, 0, 0)),
                pl.BlockSpec((L, P * DV, P * DV), lambda b: (0, 0, 0)),
                pl.BlockSpec((L, P * DV, P * DV), lambda b: (0, 0, 0)),
                pl.BlockSpec((L, 1, P * 32), lambda b: (0, 0, 0)),
                pl.BlockSpec((L, 1, P * 32), lambda b: (0, 0, 0)),
                pl.BlockSpec((P * DV, P * DV), lambda b: (0, 0)),
            ],
            out_specs=pl.BlockSpec((N_V, BB, P * DV), lambda b: (0, b, 0)),
            out_shape=jax.ShapeDtypeStruct((N_V, b4, P * DV), jnp.float32),
        )(vt, et, wp, we, wh1, wh2, ebp, hbp, ones_bd)
        return out.reshape(N_V, bc, DV).transpose(1, 0, 2)

    bc = B // NCHUNK
    outs = [run_chunk(vertices[k * bc:(k + 1) * bc],
                      edges[k * bc:(k + 1) * bc]) for k in range(NCHUNK)]
    return jnp.concatenate(outs, axis=0)
